# baseline (speedup 1.0000x reference)
import functools

import jax
import jax.numpy as jnp
from jax import lax
from jax.experimental import pallas as pl
from jax.experimental.pallas import tpu as pltpu
from jax.experimental.pallas import tpu_sc as plsc

BATCH = 16384
EMB_DIM = 64
NUM_CORES = 2
NUM_SUBCORES = 16
NUM_WORKERS = NUM_CORES * NUM_SUBCORES
ROWS_PER_W = BATCH // NUM_WORKERS
IDX_CHUNK = 128
N_CHUNKS = ROWS_PER_W // IDX_CHUNK

@functools.cache
def _sc_gather_kernel():
    mesh = plsc.VectorSubcoreMesh(core_axis_name="c", subcore_axis_name="s")

    @functools.partial(
        pl.kernel,
        mesh=mesh,
        compiler_params=pltpu.CompilerParams(use_tc_tiling_on_sc=False),
        out_type=[
            jax.ShapeDtypeStruct((BATCH, EMB_DIM), jnp.float32),
            jax.ShapeDtypeStruct((BATCH, EMB_DIM), jnp.float32),
        ],
        scratch_types=[
            pltpu.VMEM((N_CHUNKS, IDX_CHUNK), jnp.int32),
            pltpu.VMEM((ROWS_PER_W, EMB_DIM), jnp.float32),
            pltpu.VMEM((N_CHUNKS, IDX_CHUNK), jnp.int32),
            pltpu.VMEM((ROWS_PER_W, EMB_DIM), jnp.float32),
            pltpu.SemaphoreType.DMA,
        ],
    )
    def _sc_gather(uids_hbm, iids_hbm, uemb_hbm, iemb_hbm, uout_hbm, iout_hbm,
                   uidx_v, urows_v, iidx_v, irows_v, sem):
        wid = lax.axis_index("s") * NUM_CORES + lax.axis_index("c")
        base = wid * ROWS_PER_W
        pltpu.sync_copy(uids_hbm.at[pl.ds(wid * N_CHUNKS, N_CHUNKS)], uidx_v)
        pltpu.sync_copy(iids_hbm.at[pl.ds(wid * N_CHUNKS, N_CHUNKS)], iidx_v)
        copies = []
        for j in range(N_CHUNKS):
            copies.append(pltpu.async_copy(
                uemb_hbm.at[uidx_v.at[j]],
                urows_v.at[pl.ds(j * IDX_CHUNK, IDX_CHUNK)], sem))
            copies.append(pltpu.async_copy(
                iemb_hbm.at[iidx_v.at[j]],
                irows_v.at[pl.ds(j * IDX_CHUNK, IDX_CHUNK)], sem))
        for c in copies:
            c.wait()
        pltpu.sync_copy(urows_v, uout_hbm.at[pl.ds(base, ROWS_PER_W)])
        pltpu.sync_copy(irows_v, iout_hbm.at[pl.ds(base, ROWS_PER_W)])

    return _sc_gather


def _tc_body(u_ref, i_ref, wu_ref, bu_ref, wi_ref, bi_ref, o_ref):
    dn = (((1,), (1,)), ((), ()))
    u = lax.dot_general(u_ref[...], wu_ref[...], dn,
                        preferred_element_type=jnp.float32) + bu_ref[...]
    u = jnp.maximum(u, 0.0)
    i = lax.dot_general(i_ref[...], wi_ref[...], dn,
                        preferred_element_type=jnp.float32) + bi_ref[...]
    i = jnp.maximum(i, 0.0)
    un = jnp.sqrt(jnp.sum(u * u, axis=1, keepdims=True))
    inn = jnp.sqrt(jnp.sum(i * i, axis=1, keepdims=True))
    denom = jnp.maximum(un, 1e-12) * jnp.maximum(inn, 1e-12)
    o_ref[...] = jnp.sum(u * i, axis=1, keepdims=True) / denom


_TC_BLOCK = 2048


def _tc_scores(u_rows, i_rows, Wu, bu2, Wi, bi2):
    grid = (BATCH // _TC_BLOCK,)
    return pl.pallas_call(
        _tc_body,
        grid=grid,
        in_specs=[
            pl.BlockSpec((_TC_BLOCK, EMB_DIM), lambda g: (g, 0)),
            pl.BlockSpec((_TC_BLOCK, EMB_DIM), lambda g: (g, 0)),
            pl.BlockSpec((EMB_DIM, EMB_DIM), lambda g: (0, 0)),
            pl.BlockSpec((1, EMB_DIM), lambda g: (0, 0)),
            pl.BlockSpec((EMB_DIM, EMB_DIM), lambda g: (0, 0)),
            pl.BlockSpec((1, EMB_DIM), lambda g: (0, 0)),
        ],
        out_specs=pl.BlockSpec((_TC_BLOCK, 1), lambda g: (g, 0)),
        out_shape=jax.ShapeDtypeStruct((BATCH, 1), jnp.float32),
    )(u_rows, i_rows, Wu, bu2, Wi, bi2)


def kernel(user_ids, item_ids, user_emb, item_emb, Wu, bu, Wi, bi):
    uids = user_ids.astype(jnp.int32).reshape(BATCH // IDX_CHUNK, IDX_CHUNK)
    iids = item_ids.astype(jnp.int32).reshape(BATCH // IDX_CHUNK, IDX_CHUNK)
    u_rows, i_rows = _sc_gather_kernel()(uids, iids, user_emb, item_emb)
    scores = _tc_scores(u_rows, i_rows, Wu, bu.reshape(1, EMB_DIM),
                        Wi, bi.reshape(1, EMB_DIM))
    return scores.reshape(BATCH)
